# baseline (speedup 1.0000x reference)
import jax
import jax.numpy as jnp
from jax.experimental import pallas as pl


def _combine(tp_ref, sp_ref, tm_ref, sm_ref, em_ref, ex_ref, o_ref):
    o_ref[...] = (tp_ref[...] * tm_ref[...] + sp_ref[...] * sm_ref[...]
                  + ex_ref[...] * em_ref[...])


def kernel(input_ids, position_ids, exaggeration, text_emb, text_pos_emb,
           speech_emb, speech_pos_emb, W_exag, b_exag):
    Bb, Ss = input_ids.shape
    idx = jnp.broadcast_to(jnp.arange(Ss, dtype=input_ids.dtype)[None, :], (Bb, Ss))
    is_zero = input_ids == 0
    has_zero = jnp.any(is_zero, axis=1)
    first_zero = jnp.argmax(is_zero.astype(jnp.int32), axis=1).astype(input_ids.dtype)
    zero_pos = jnp.where(has_zero, first_zero, jnp.full_like(first_zero, -1))
    exag_mask = input_ids == 2047
    base_text_mask = (idx <= zero_pos[:, None]) & has_zero[:, None]
    text_mask = base_text_mask & (~exag_mask)
    speech_mask = (~base_text_mask) & (~exag_mask)
    zero_idx = jnp.zeros_like(input_ids)
    safe_text_ids = jnp.where(text_mask, input_ids, zero_idx)
    safe_speech_ids = jnp.where(speech_mask, input_ids, zero_idx)
    text_pos_ids = position_ids * text_mask.astype(position_ids.dtype)
    speech_pos_ids = position_ids * speech_mask.astype(position_ids.dtype)
    text_part = jnp.take(text_emb, safe_text_ids, axis=0) + jnp.take(text_pos_emb, text_pos_ids, axis=0)
    speech_part = jnp.take(speech_emb, safe_speech_ids, axis=0) + jnp.take(speech_pos_emb, speech_pos_ids, axis=0)
    row_ids = jnp.arange(Bb)
    cfg = jnp.where(row_ids == 1, 0.0, 1.0).reshape(Bb, 1, 1).astype(jnp.float32)
    text_part = text_part * cfg
    exag_embed = (exaggeration.reshape(1, 1) @ W_exag.T + b_exag).reshape(1, 1, -1)
    tm = text_mask.astype(jnp.float32)[..., None]
    sm = speech_mask.astype(jnp.float32)[..., None]
    em = exag_mask.astype(jnp.float32)[..., None]
    ex = jnp.broadcast_to(exag_embed, text_part.shape)
    out = pl.pallas_call(
        _combine,
        out_shape=jax.ShapeDtypeStruct(text_part.shape, jnp.float32),
        grid=(Bb * Ss // 512,),
        in_specs=[pl.BlockSpec((1, 512, 1024), lambda i: (0, i, 0))] * 6,
        out_specs=pl.BlockSpec((1, 512, 1024), lambda i: (0, i, 0)),
    )(text_part.reshape(1, Bb * Ss, -1), speech_part.reshape(1, Bb * Ss, -1),
      jnp.broadcast_to(tm, text_part.shape).reshape(1, Bb * Ss, -1),
      jnp.broadcast_to(sm, text_part.shape).reshape(1, Bb * Ss, -1),
      jnp.broadcast_to(em, text_part.shape).reshape(1, Bb * Ss, -1),
      ex.reshape(1, Bb * Ss, -1))
    return out.reshape(Bb, Ss, -1)


# speedup vs baseline: 2.3250x; 2.3250x over previous
import functools

import jax
import jax.numpy as jnp
from jax import lax
from jax.experimental import pallas as pl
from jax.experimental.pallas import tpu as pltpu
from jax.experimental.pallas import tpu_sc as plsc

_EXAG_TOK = 2047
_DIM = 1024
_NC, _NS, _L = 2, 16, 16
_TPW = 256
_NBUF = 8
_BIG = 1 << 30


def _body(ids_hbm, pos_hbm, exag_hbm, wcol_hbm, bias_hbm,
          text_emb, text_pos, speech_emb, speech_pos, out_hbm,
          ids_v, pos_v, cls_v, wv, bv, exag_v, crow_v,
          ra_v, rb_v, obuf_v, ina, inb, osem):
    c = lax.axis_index("c")
    s = lax.axis_index("s")
    srow = _NS * _TPW
    row_base = s * _TPW
    base = c * srow + row_base

    pltpu.sync_copy(ids_hbm.at[pl.ds(c * srow, srow)], ids_v.at[pl.ds(0, srow)])
    pltpu.sync_copy(pos_hbm.at[pl.ds(base, _TPW)], pos_v.at[pl.ds(0, _TPW)])
    pltpu.sync_copy(wcol_hbm, wv)
    pltpu.sync_copy(bias_hbm, bv)
    pltpu.sync_copy(exag_hbm, exag_v)

    ev = exag_v[...]
    for i in range(_DIM // _L):
        d = pl.ds(i * _L, _L)
        crow_v[0, d] = jnp.zeros((_L,), jnp.float32)
        crow_v[1, d] = ev * wv[d] + bv[d]

    iota = lax.iota(jnp.int32, _L)

    def scan_step(j, acc):
        v = ids_v[pl.ds(j * _L, _L)]
        t = j * _L + iota
        return jnp.minimum(acc, jnp.where(v == 0, t, _BIG))

    m = lax.fori_loop(0, srow // _L, scan_step,
                      jnp.full((_L,), _BIG, jnp.int32))
    for sh in (1, 2, 4, 8):
        m = jnp.minimum(m, jnp.take_along_axis(m, iota ^ sh, axis=0))

    rvec = jnp.full((_L,), c, jnp.int32)
    for j in range(_TPW // _L):
        v = ids_v[pl.ds(row_base + j * _L, _L)]
        t = row_base + j * _L + iota
        bt = (t <= m) & (m < _BIG)
        cls = jnp.where(v == _EXAG_TOK, 2,
                        jnp.where(bt, jnp.where(rvec == 1, 3, 0), 1))
        cls_v[pl.ds(j * _L, _L)] = cls

    def issue(cls, tid, tpos, slot):
        @pl.when(cls == 0)
        def _():
            pltpu.async_copy(text_emb.at[pl.ds(tid, 1)],
                             ra_v.at[pl.ds(slot, 1)], ina.at[slot])
            pltpu.async_copy(text_pos.at[pl.ds(tpos, 1)],
                             rb_v.at[pl.ds(slot, 1)], inb.at[slot])

        @pl.when(cls == 1)
        def _():
            pltpu.async_copy(speech_emb.at[pl.ds(tid, 1)],
                             ra_v.at[pl.ds(slot, 1)], ina.at[slot])
            pltpu.async_copy(speech_pos.at[pl.ds(tpos, 1)],
                             rb_v.at[pl.ds(slot, 1)], inb.at[slot])

    def complete(cls, slot, pong):
        @pl.when(cls < 2)
        def _():
            pltpu.make_async_copy(text_emb.at[pl.ds(0, 1)],
                                  ra_v.at[pl.ds(slot, 1)], ina.at[slot]).wait()
            pltpu.make_async_copy(text_emb.at[pl.ds(0, 1)],
                                  rb_v.at[pl.ds(slot, 1)], inb.at[slot]).wait()
            for i in range(_DIM // _L):
                d = pl.ds(i * _L, _L)
                obuf_v[pong, slot, d] = ra_v[slot, d] + rb_v[slot, d]

        @pl.when(cls == 2)
        def _():
            for i in range(_DIM // _L):
                d = pl.ds(i * _L, _L)
                obuf_v[pong, slot, d] = crow_v[1, d]

        @pl.when(cls == 3)
        def _():
            for i in range(_DIM // _L):
                d = pl.ds(i * _L, _L)
                obuf_v[pong, slot, d] = crow_v[0, d]

    idv0 = ids_v[pl.ds(row_base, _L)]
    pov0 = pos_v[pl.ds(0, _L)]
    clv0 = cls_v[pl.ds(0, _L)]
    for b in range(_NBUF):
        issue(clv0[b], idv0[b], pov0[b], b)

    def turn(g, pong):
        idv = ids_v[pl.ds(row_base + g * _NBUF, _L)]
        pov = pos_v[pl.ds(g * _NBUF, _L)]
        clv = cls_v[pl.ds(g * _NBUF, _L)]
        @pl.when(g >= 2)
        def _():
            pltpu.make_async_copy(out_hbm.at[pl.ds(0, _NBUF)],
                                  obuf_v.at[pong], osem.at[pong]).wait()
        for b in range(_NBUF):
            k = g * _NBUF + b
            complete(clv[b], b, pong)
            @pl.when(k + _NBUF < _TPW)
            def _():
                issue(clv[b + _NBUF], idv[b + _NBUF], pov[b + _NBUF], b)
        pltpu.async_copy(obuf_v.at[pong], out_hbm.at[pl.ds(base + g * _NBUF, _NBUF)],
                         osem.at[pong])

    def outer(gg, _):
        turn(2 * gg, 0)
        turn(2 * gg + 1, 1)
        return 0

    lax.fori_loop(0, _TPW // (2 * _NBUF), outer, 0)

    for pong in range(2):
        pltpu.make_async_copy(out_hbm.at[pl.ds(0, _NBUF)],
                              obuf_v.at[pong], osem.at[pong]).wait()


@jax.jit
def _run(ids, pos, exag16, wcol, bias, text_emb, text_pos, speech_emb, speech_pos):
    n = ids.shape[0]
    mesh = plsc.VectorSubcoreMesh(core_axis_name="c", subcore_axis_name="s")
    f = functools.partial(
        pl.kernel, mesh=mesh,
        out_type=jax.ShapeDtypeStruct((n, _DIM), jnp.float32),
        scratch_types=[
            pltpu.VMEM((_NS * _TPW + _L,), jnp.int32),
            pltpu.VMEM((_TPW + _L,), jnp.int32),
            pltpu.VMEM((_TPW + _L,), jnp.int32),
            pltpu.VMEM((_DIM,), jnp.float32),
            pltpu.VMEM((_DIM,), jnp.float32),
            pltpu.VMEM((_L,), jnp.float32),
            pltpu.VMEM((2, _DIM), jnp.float32),
            pltpu.VMEM((_NBUF, _DIM), jnp.float32),
            pltpu.VMEM((_NBUF, _DIM), jnp.float32),
            pltpu.VMEM((2, _NBUF, _DIM), jnp.float32),
            pltpu.SemaphoreType.DMA((_NBUF,)),
            pltpu.SemaphoreType.DMA((_NBUF,)),
            pltpu.SemaphoreType.DMA((2,)),
        ],
    )(_body)
    return f(ids, pos, exag16, wcol, bias, text_emb, text_pos, speech_emb, speech_pos)


def kernel(input_ids, position_ids, exaggeration, text_emb, text_pos_emb,
           speech_emb, speech_pos_emb, W_exag, b_exag):
    Bb, Ss = input_ids.shape
    ids = input_ids.reshape(-1).astype(jnp.int32)
    pos = position_ids.reshape(-1).astype(jnp.int32)
    exag16 = jnp.broadcast_to(exaggeration.astype(jnp.float32), (_L,))
    wcol = W_exag[:, 0].astype(jnp.float32)
    out = _run(ids, pos, exag16, wcol, b_exag.astype(jnp.float32),
               text_emb, text_pos_emb, speech_emb, speech_pos_emb)
    return out.reshape(Bb, Ss, _DIM)


# speedup vs baseline: 3.2547x; 1.3999x over previous
import functools

import jax
import jax.numpy as jnp
from jax import lax
from jax.experimental import pallas as pl
from jax.experimental.pallas import tpu as pltpu
from jax.experimental.pallas import tpu_sc as plsc

_EXAG_TOK = 2047
_DIM = 1024
_NC, _NS, _L = 2, 16, 16
_TPW = 256
_CH = 16
_NCHK = _TPW // _CH
_BIG = 1 << 30
_UNR = 4


def _body(ids_hbm, pos_hbm, exag_hbm, wcol_hbm, bias_hbm,
          text_emb, text_pos, speech_emb, speech_pos, out_hbm,
          ids_v, pos_v, wv, bv, exag_v, erow_v, zrow_v,
          ra_v, rb_v, gsa, gsb, ssem):
    c = lax.axis_index("c")
    s = lax.axis_index("s")
    srow = _NS * _TPW
    row_base = s * _TPW
    base = c * srow + row_base

    pltpu.sync_copy(ids_hbm.at[pl.ds(c * srow, srow)], ids_v.at[pl.ds(0, srow)])
    pltpu.sync_copy(pos_hbm.at[pl.ds(base, _TPW)], pos_v)
    pltpu.sync_copy(wcol_hbm, wv)
    pltpu.sync_copy(bias_hbm, bv)
    pltpu.sync_copy(exag_hbm, exag_v)

    ev = exag_v[...]
    zv = jnp.zeros((_L,), jnp.float32)

    def const_step(i, _):
        d = pl.ds(i * _L, _L)
        erow_v[0, d] = ev * wv[d] + bv[d]
        for r in range(_CH):
            zrow_v[r, d] = zv
        return 0

    lax.fori_loop(0, _DIM // _L, const_step, 0)

    iota = lax.iota(jnp.int32, _L)

    def scan_step(j, acc):
        v = ids_v[pl.ds(j * _L, _L)]
        t = j * _L + iota
        return jnp.minimum(acc, jnp.where(v == 0, t, _BIG))

    m = lax.fori_loop(0, srow // _L, scan_step,
                      jnp.full((_L,), _BIG, jnp.int32))
    for sh in (1, 2, 4, 8):
        m = jnp.minimum(m, jnp.take_along_axis(m, iota ^ sh, axis=0))
    zp = m[0]
    has_zero = zp < _BIG
    b0 = jnp.where(has_zero,
                   jnp.clip(zp + 1 - row_base, 0, _TPW), 0).astype(jnp.int32)
    nfull = b0 // _CH
    rem = b0 - nfull * _CH
    sb = nfull + jnp.where(rem > 0, 1, 0)

    def wait_out(par):
        pltpu.make_async_copy(ra_v.at[par], out_hbm.at[pl.ds(0, _CH)],
                              ssem.at[par]).wait()

    def add_rows(par, dst_par=None):
        dpar = par if dst_par is None else dst_par
        for r in range(_CH):
            def add_step(i, _, r=r):
                for u in range(_UNR):
                    d = pl.ds((i * _UNR + u) * _L, _L)
                    ra_v[dpar, r, d] = ra_v[par, r, d] + rb_v[par, r, d]
                return 0
            lax.fori_loop(0, _DIM // (_L * _UNR), add_step, 0)

    def run_span(tab_a, tab_b, start, end):
        nch = end - start

        def fire(j, par):
            @pl.when(j >= 2)
            def _():
                wait_out(par)
            ci = start + j
            tk = pl.ds(row_base + ci * _CH, _CH)
            pltpu.async_copy(tab_a.at[ids_v.at[tk]], ra_v.at[par], gsa.at[par])
            pltpu.async_copy(tab_b.at[pos_v.at[pl.ds(ci * _CH, _CH)]],
                             rb_v.at[par], gsb.at[par])

        def finish(j, par):
            pltpu.make_async_copy(text_emb.at[pl.ds(0, _CH)], ra_v.at[par],
                                  gsa.at[par]).wait()
            pltpu.make_async_copy(text_emb.at[pl.ds(0, _CH)], rb_v.at[par],
                                  gsb.at[par]).wait()
            add_rows(par)
            ci = start + j
            pltpu.async_copy(ra_v.at[par],
                             out_hbm.at[pl.ds(base + ci * _CH, _CH)],
                             ssem.at[par])

        def pair_step(cc, _):
            j0 = 2 * cc
            fire(j0, 0)
            @pl.when(j0 >= 1)
            def _():
                finish(j0 - 1, 1)
            @pl.when(j0 + 1 < nch)
            def _():
                fire(j0 + 1, 1)
            finish(j0, 0)
            return 0

        lax.fori_loop(0, (nch + 1) // 2, pair_step, 0)
        @pl.when((nch >= 2) & (nch % 2 == 0))
        def _():
            finish(nch - 1, 1)
        @pl.when(nch >= 1)
        def _():
            wait_out(0)
        @pl.when(nch >= 2)
        def _():
            wait_out(1)

    @pl.when(c == 0)
    def _():
        run_span(text_emb, text_pos, jnp.int32(0), nfull)

    @pl.when(c == 1)
    def _():
        def zpair(cc, _):
            for par in range(2):
                j = 2 * cc + par
                @pl.when(j < nfull)
                def _():
                    @pl.when(j >= 2)
                    def _():
                        pltpu.make_async_copy(zrow_v, out_hbm.at[pl.ds(0, _CH)],
                                              ssem.at[par]).wait()
                    pltpu.async_copy(zrow_v,
                                     out_hbm.at[pl.ds(base + j * _CH, _CH)],
                                     ssem.at[par])
            return 0

        lax.fori_loop(0, (nfull + 1) // 2, zpair, 0)
        for par in range(2):
            @pl.when(nfull >= par + 1)
            def _():
                pltpu.make_async_copy(zrow_v, out_hbm.at[pl.ds(0, _CH)],
                                      ssem.at[par]).wait()

    @pl.when(rem > 0)
    def _():
        ci = nfull
        tk = pl.ds(row_base + ci * _CH, _CH)
        pk = pl.ds(ci * _CH, _CH)
        pltpu.async_copy(speech_emb.at[ids_v.at[tk]], ra_v.at[1], gsa.at[1])
        pltpu.async_copy(speech_pos.at[pos_v.at[pk]], rb_v.at[1], gsb.at[1])
        pltpu.make_async_copy(text_emb.at[pl.ds(0, _CH)], ra_v.at[1],
                              gsa.at[1]).wait()
        pltpu.make_async_copy(text_emb.at[pl.ds(0, _CH)], rb_v.at[1],
                              gsb.at[1]).wait()
        add_rows(1)

        @pl.when(c == 0)
        def _():
            pltpu.async_copy(text_emb.at[ids_v.at[tk]], ra_v.at[0], gsa.at[0])
            pltpu.async_copy(text_pos.at[pos_v.at[pk]], rb_v.at[0], gsb.at[0])
            pltpu.make_async_copy(text_emb.at[pl.ds(0, _CH)], ra_v.at[0],
                                  gsa.at[0]).wait()
            pltpu.make_async_copy(text_emb.at[pl.ds(0, _CH)], rb_v.at[0],
                                  gsb.at[0]).wait()
            add_rows(0)
            for r in range(_CH):
                @pl.when(r >= rem)
                def _(r=r):
                    def cp(i, _):
                        for u in range(_UNR):
                            d = pl.ds((i * _UNR + u) * _L, _L)
                            ra_v[0, r, d] = ra_v[1, r, d]
                        return 0
                    lax.fori_loop(0, _DIM // (_L * _UNR), cp, 0)

        @pl.when(c == 1)
        def _():
            for r in range(_CH):
                @pl.when(r < rem)
                def _(r=r):
                    def cp(i, _):
                        for u in range(_UNR):
                            d = pl.ds((i * _UNR + u) * _L, _L)
                            ra_v[1, r, d] = zrow_v[r, d]
                        return 0
                    lax.fori_loop(0, _DIM // (_L * _UNR), cp, 0)

        src = 0
        @pl.when(c == 0)
        def _():
            pltpu.async_copy(ra_v.at[0], out_hbm.at[pl.ds(base + ci * _CH, _CH)],
                             ssem.at[0])
            wait_out(0)
        @pl.when(c == 1)
        def _():
            pltpu.async_copy(ra_v.at[1], out_hbm.at[pl.ds(base + ci * _CH, _CH)],
                             ssem.at[1])
            wait_out(1)

    run_span(speech_emb, speech_pos, sb, jnp.int32(_NCHK))

    def exag_step(j, _):
        v = ids_v[pl.ds(row_base + j * _L, _L)]
        exm = jnp.where(v == _EXAG_TOK, 1, 0)
        any_v = exm
        for sh in (1, 2, 4, 8):
            any_v = jnp.maximum(any_v,
                                jnp.take_along_axis(any_v, iota ^ sh, axis=0))
        @pl.when(any_v[0] > 0)
        def _():
            for lane in range(_L):
                @pl.when(exm[lane] > 0)
                def _(lane=lane):
                    pltpu.sync_copy(
                        erow_v,
                        out_hbm.at[pl.ds(base + j * _L + lane, 1)])
        return 0

    lax.fori_loop(0, _TPW // _L, exag_step, 0)


@jax.jit
def _run(ids, pos, exag16, wcol, bias, text_emb, text_pos, speech_emb, speech_pos):
    n = ids.shape[0]
    mesh = plsc.VectorSubcoreMesh(core_axis_name="c", subcore_axis_name="s")
    f = functools.partial(
        pl.kernel, mesh=mesh,
        out_type=jax.ShapeDtypeStruct((n, _DIM), jnp.float32),
        scratch_types=[
            pltpu.VMEM((_NS * _TPW + _L,), jnp.int32),
            pltpu.VMEM((_TPW,), jnp.int32),
            pltpu.VMEM((_DIM,), jnp.float32),
            pltpu.VMEM((_DIM,), jnp.float32),
            pltpu.VMEM((_L,), jnp.float32),
            pltpu.VMEM((1, _DIM), jnp.float32),
            pltpu.VMEM((_CH, _DIM), jnp.float32),
            pltpu.VMEM((2, _CH, _DIM), jnp.float32),
            pltpu.VMEM((2, _CH, _DIM), jnp.float32),
            pltpu.SemaphoreType.DMA((2,)),
            pltpu.SemaphoreType.DMA((2,)),
            pltpu.SemaphoreType.DMA((2,)),
        ],
    )(_body)
    return f(ids, pos, exag16, wcol, bias, text_emb, text_pos, speech_emb, speech_pos)


def kernel(input_ids, position_ids, exaggeration, text_emb, text_pos_emb,
           speech_emb, speech_pos_emb, W_exag, b_exag):
    Bb, Ss = input_ids.shape
    ids = input_ids.reshape(-1).astype(jnp.int32)
    pos = position_ids.reshape(-1).astype(jnp.int32)
    exag16 = jnp.broadcast_to(exaggeration.astype(jnp.float32), (_L,))
    wcol = W_exag[:, 0].astype(jnp.float32)
    out = _run(ids, pos, exag16, wcol, b_exag.astype(jnp.float32),
               text_emb, text_pos_emb, speech_emb, speech_pos_emb)
    return out.reshape(Bb, Ss, _DIM)


# speedup vs baseline: 3.7026x; 1.1376x over previous
import functools

import jax
import jax.numpy as jnp
from jax import lax
from jax.experimental import pallas as pl
from jax.experimental.pallas import tpu as pltpu
from jax.experimental.pallas import tpu_sc as plsc

_EXAG_TOK = 2047
_DIM = 1024
_NC, _NS, _L = 2, 16, 16
_TPW = 256
_CH = 8
_NCHK = _TPW // _CH
_NP = 4
_BIG = 1 << 30
_UNR = 4


def _body(ids_hbm, pos_hbm, exag_hbm, wcol_hbm, bias_hbm,
          text_emb, text_pos, speech_emb, speech_pos, out_hbm,
          ids_v, pos_v, wv, bv, exag_v, erow_v, zrow_v,
          ra_v, rb_v, ob_v, gsa, gsb, ssem):
    c = lax.axis_index("c")
    s = lax.axis_index("s")
    srow = _NS * _TPW
    row_base = s * _TPW
    base = c * srow + row_base

    pltpu.sync_copy(ids_hbm.at[pl.ds(c * srow, srow)], ids_v.at[pl.ds(0, srow)])
    pltpu.sync_copy(pos_hbm.at[pl.ds(base, _TPW)], pos_v)
    pltpu.sync_copy(wcol_hbm, wv)
    pltpu.sync_copy(bias_hbm, bv)
    pltpu.sync_copy(exag_hbm, exag_v)

    ev = exag_v[...]
    zv = jnp.zeros((_L,), jnp.float32)

    def const_step(i, _):
        d = pl.ds(i * _L, _L)
        erow_v[0, d] = ev * wv[d] + bv[d]
        for r in range(_CH):
            zrow_v[r, d] = zv
        return 0

    lax.fori_loop(0, _DIM // _L, const_step, 0)

    iota = lax.iota(jnp.int32, _L)

    def scan_step(j, acc):
        v = ids_v[pl.ds(j * _L, _L)]
        t = j * _L + iota
        return jnp.minimum(acc, jnp.where(v == 0, t, _BIG))

    m = lax.fori_loop(0, srow // _L, scan_step,
                      jnp.full((_L,), _BIG, jnp.int32))
    for sh in (1, 2, 4, 8):
        m = jnp.minimum(m, jnp.take_along_axis(m, iota ^ sh, axis=0))
    zp = m[0]
    has_zero = zp < _BIG
    b0 = jnp.where(has_zero,
                   jnp.clip(zp + 1 - row_base, 0, _TPW), 0).astype(jnp.int32)
    nfull = b0 // _CH
    rem = b0 - nfull * _CH
    sb = nfull + jnp.where(rem > 0, 1, 0)

    def wait_out(par):
        pltpu.make_async_copy(ob_v.at[par], out_hbm.at[pl.ds(0, _CH)],
                              ssem.at[par]).wait()

    def add_rows(par):
        for r in range(_CH):
            def add_step(i, _, r=r):
                for u in range(_UNR):
                    d = pl.ds((i * _UNR + u) * _L, _L)
                    ob_v[par, r, d] = ra_v[par, r, d] + rb_v[par, r, d]
                return 0
            lax.fori_loop(0, _DIM // (_L * _UNR), add_step, 0)

    def run_span(tab_a, tab_b, start, end):
        nch = end - start

        def fire(j, par):
            ci = start + j
            tk = pl.ds(row_base + ci * _CH, _CH)
            pltpu.async_copy(tab_a.at[ids_v.at[tk]], ra_v.at[par], gsa.at[par])
            pltpu.async_copy(tab_b.at[pos_v.at[pl.ds(ci * _CH, _CH)]],
                             rb_v.at[par], gsb.at[par])

        def finish(j, par):
            pltpu.make_async_copy(text_emb.at[pl.ds(0, _CH)], ra_v.at[par],
                                  gsa.at[par]).wait()
            pltpu.make_async_copy(text_emb.at[pl.ds(0, _CH)], rb_v.at[par],
                                  gsb.at[par]).wait()
            @pl.when(j >= _NP)
            def _():
                wait_out(par)
            add_rows(par)
            ci = start + j
            pltpu.async_copy(ob_v.at[par],
                             out_hbm.at[pl.ds(base + ci * _CH, _CH)],
                             ssem.at[par])

        for p in range(_NP - 1):
            @pl.when(p < nch)
            def _(p=p):
                fire(jnp.int32(p), p)

        def quad_step(cc, _):
            for par in range(_NP):
                j = _NP * cc + par
                @pl.when(j < nch)
                def _(j=j, par=par):
                    @pl.when(j + _NP - 1 < nch)
                    def _():
                        fire(j + _NP - 1, (par + _NP - 1) % _NP)
                    finish(j, par)
            return 0

        lax.fori_loop(0, (nch + _NP - 1) // _NP, quad_step, 0)
        for p in range(_NP):
            @pl.when(nch >= p + 1)
            def _(p=p):
                wait_out(p)

    @pl.when(c == 0)
    def _():
        run_span(text_emb, text_pos, jnp.int32(0), nfull)

    @pl.when(c == 1)
    def _():
        def zpair(cc, _):
            for par in range(2):
                j = 2 * cc + par
                @pl.when(j < nfull)
                def _():
                    @pl.when(j >= 2)
                    def _():
                        pltpu.make_async_copy(zrow_v, out_hbm.at[pl.ds(0, _CH)],
                                              ssem.at[par]).wait()
                    pltpu.async_copy(zrow_v,
                                     out_hbm.at[pl.ds(base + j * _CH, _CH)],
                                     ssem.at[par])
            return 0

        lax.fori_loop(0, (nfull + 1) // 2, zpair, 0)
        for par in range(2):
            @pl.when(nfull >= par + 1)
            def _():
                pltpu.make_async_copy(zrow_v, out_hbm.at[pl.ds(0, _CH)],
                                      ssem.at[par]).wait()

    @pl.when(rem > 0)
    def _():
        ci = nfull
        tk = pl.ds(row_base + ci * _CH, _CH)
        pk = pl.ds(ci * _CH, _CH)
        pltpu.async_copy(speech_emb.at[ids_v.at[tk]], ra_v.at[1], gsa.at[1])
        pltpu.async_copy(speech_pos.at[pos_v.at[pk]], rb_v.at[1], gsb.at[1])
        pltpu.make_async_copy(text_emb.at[pl.ds(0, _CH)], ra_v.at[1],
                              gsa.at[1]).wait()
        pltpu.make_async_copy(text_emb.at[pl.ds(0, _CH)], rb_v.at[1],
                              gsb.at[1]).wait()
        add_rows(1)

        @pl.when(c == 0)
        def _():
            pltpu.async_copy(text_emb.at[ids_v.at[tk]], ra_v.at[0], gsa.at[0])
            pltpu.async_copy(text_pos.at[pos_v.at[pk]], rb_v.at[0], gsb.at[0])
            pltpu.make_async_copy(text_emb.at[pl.ds(0, _CH)], ra_v.at[0],
                                  gsa.at[0]).wait()
            pltpu.make_async_copy(text_emb.at[pl.ds(0, _CH)], rb_v.at[0],
                                  gsb.at[0]).wait()
            add_rows(0)
            for r in range(_CH):
                @pl.when(r >= rem)
                def _(r=r):
                    def cp(i, _):
                        for u in range(_UNR):
                            d = pl.ds((i * _UNR + u) * _L, _L)
                            ob_v[0, r, d] = ob_v[1, r, d]
                        return 0
                    lax.fori_loop(0, _DIM // (_L * _UNR), cp, 0)

        @pl.when(c == 1)
        def _():
            for r in range(_CH):
                @pl.when(r < rem)
                def _(r=r):
                    def cp(i, _):
                        for u in range(_UNR):
                            d = pl.ds((i * _UNR + u) * _L, _L)
                            ob_v[1, r, d] = zrow_v[r, d]
                        return 0
                    lax.fori_loop(0, _DIM // (_L * _UNR), cp, 0)

        @pl.when(c == 0)
        def _():
            pltpu.async_copy(ob_v.at[0], out_hbm.at[pl.ds(base + ci * _CH, _CH)],
                             ssem.at[0])
            wait_out(0)
        @pl.when(c == 1)
        def _():
            pltpu.async_copy(ob_v.at[1], out_hbm.at[pl.ds(base + ci * _CH, _CH)],
                             ssem.at[1])
            wait_out(1)

    run_span(speech_emb, speech_pos, sb, jnp.int32(_NCHK))

    def exag_step(j, _):
        v = ids_v[pl.ds(row_base + j * _L, _L)]
        exm = jnp.where(v == _EXAG_TOK, 1, 0)
        any_v = exm
        for sh in (1, 2, 4, 8):
            any_v = jnp.maximum(any_v,
                                jnp.take_along_axis(any_v, iota ^ sh, axis=0))
        @pl.when(any_v[0] > 0)
        def _():
            for lane in range(_L):
                @pl.when(exm[lane] > 0)
                def _(lane=lane):
                    pltpu.sync_copy(
                        erow_v,
                        out_hbm.at[pl.ds(base + j * _L + lane, 1)])
        return 0

    lax.fori_loop(0, _TPW // _L, exag_step, 0)


@jax.jit
def _run(ids, pos, exag16, wcol, bias, text_emb, text_pos, speech_emb, speech_pos):
    n = ids.shape[0]
    mesh = plsc.VectorSubcoreMesh(core_axis_name="c", subcore_axis_name="s")
    f = functools.partial(
        pl.kernel, mesh=mesh,
        out_type=jax.ShapeDtypeStruct((n, _DIM), jnp.float32),
        scratch_types=[
            pltpu.VMEM((_NS * _TPW + _L,), jnp.int32),
            pltpu.VMEM((_TPW,), jnp.int32),
            pltpu.VMEM((_DIM,), jnp.float32),
            pltpu.VMEM((_DIM,), jnp.float32),
            pltpu.VMEM((_L,), jnp.float32),
            pltpu.VMEM((1, _DIM), jnp.float32),
            pltpu.VMEM((_CH, _DIM), jnp.float32),
            pltpu.VMEM((_NP, _CH, _DIM), jnp.float32),
            pltpu.VMEM((_NP, _CH, _DIM), jnp.float32),
            pltpu.VMEM((_NP, _CH, _DIM), jnp.float32),
            pltpu.SemaphoreType.DMA((_NP,)),
            pltpu.SemaphoreType.DMA((_NP,)),
            pltpu.SemaphoreType.DMA((_NP,)),
        ],
    )(_body)
    return f(ids, pos, exag16, wcol, bias, text_emb, text_pos, speech_emb, speech_pos)


def kernel(input_ids, position_ids, exaggeration, text_emb, text_pos_emb,
           speech_emb, speech_pos_emb, W_exag, b_exag):
    Bb, Ss = input_ids.shape
    ids = input_ids.reshape(-1).astype(jnp.int32)
    pos = position_ids.reshape(-1).astype(jnp.int32)
    exag16 = jnp.broadcast_to(exaggeration.astype(jnp.float32), (_L,))
    wcol = W_exag[:, 0].astype(jnp.float32)
    out = _run(ids, pos, exag16, wcol, b_exag.astype(jnp.float32),
               text_emb, text_pos_emb, speech_emb, speech_pos_emb)
    return out.reshape(Bb, Ss, _DIM)


# speedup vs baseline: 3.7689x; 1.0179x over previous
import functools

import jax
import jax.numpy as jnp
from jax import lax
from jax.experimental import pallas as pl
from jax.experimental.pallas import tpu as pltpu
from jax.experimental.pallas import tpu_sc as plsc

_EXAG_TOK = 2047
_DIM = 1024
_NC, _NS, _L = 2, 16, 16
_TPW = 256
_CH = 8
_NCHK = _TPW // _CH
_NP = 4
_BIG = 1 << 30
_UNR = 4


def _body(ids_hbm, pos_hbm, exag_hbm, wcol_hbm, bias_hbm,
          text_emb, text_pos, speech_emb, speech_pos, out_hbm,
          ids_v, pos_v, wv, bv, exag_v, erow_v, zrow_v,
          ra_v, rb_v, ob_v, gsa, gsb, ssem):
    c = lax.axis_index("c")
    s = lax.axis_index("s")
    wid = s * _NC + c
    row = wid // _NS
    srow = _NS * _TPW
    row_base = (wid % _NS) * _TPW
    base = row * srow + row_base

    pltpu.sync_copy(ids_hbm.at[pl.ds(row * srow, srow)], ids_v.at[pl.ds(0, srow)])
    pltpu.sync_copy(pos_hbm.at[pl.ds(base, _TPW)], pos_v)
    pltpu.sync_copy(wcol_hbm, wv)
    pltpu.sync_copy(bias_hbm, bv)
    pltpu.sync_copy(exag_hbm, exag_v)

    ev = exag_v[...]
    zv = jnp.zeros((_L,), jnp.float32)

    def const_step(i, _):
        d = pl.ds(i * _L, _L)
        erow_v[0, d] = ev * wv[d] + bv[d]
        for r in range(_CH):
            zrow_v[r, d] = zv
        return 0

    lax.fori_loop(0, _DIM // _L, const_step, 0)

    iota = lax.iota(jnp.int32, _L)

    def scan_step(j, acc):
        v = ids_v[pl.ds(j * _L, _L)]
        t = j * _L + iota
        return jnp.minimum(acc, jnp.where(v == 0, t, _BIG))

    m = lax.fori_loop(0, srow // _L, scan_step,
                      jnp.full((_L,), _BIG, jnp.int32))
    for sh in (1, 2, 4, 8):
        m = jnp.minimum(m, jnp.take_along_axis(m, iota ^ sh, axis=0))
    zp = m[0]
    has_zero = zp < _BIG
    b0 = jnp.where(has_zero,
                   jnp.clip(zp + 1 - row_base, 0, _TPW), 0).astype(jnp.int32)
    nfull = b0 // _CH
    rem = b0 - nfull * _CH
    sb = nfull + jnp.where(rem > 0, 1, 0)

    def wait_out(par):
        pltpu.make_async_copy(ob_v.at[par], out_hbm.at[pl.ds(0, _CH)],
                              ssem.at[par]).wait()

    def add_rows(par):
        for r in range(_CH):
            def add_step(i, _, r=r):
                for u in range(_UNR):
                    d = pl.ds((i * _UNR + u) * _L, _L)
                    ob_v[par, r, d] = ra_v[par, r, d] + rb_v[par, r, d]
                return 0
            lax.fori_loop(0, _DIM // (_L * _UNR), add_step, 0)

    def run_span(tab_a, tab_b, start, end):
        nch = end - start

        def fire(j, par):
            ci = start + j
            tk = pl.ds(row_base + ci * _CH, _CH)
            pltpu.async_copy(tab_a.at[ids_v.at[tk]], ra_v.at[par], gsa.at[par])
            pltpu.async_copy(tab_b.at[pos_v.at[pl.ds(ci * _CH, _CH)]],
                             rb_v.at[par], gsb.at[par])

        def finish(j, par):
            pltpu.make_async_copy(text_emb.at[pl.ds(0, _CH)], ra_v.at[par],
                                  gsa.at[par]).wait()
            pltpu.make_async_copy(text_emb.at[pl.ds(0, _CH)], rb_v.at[par],
                                  gsb.at[par]).wait()
            @pl.when(j >= _NP)
            def _():
                wait_out(par)
            add_rows(par)
            ci = start + j
            pltpu.async_copy(ob_v.at[par],
                             out_hbm.at[pl.ds(base + ci * _CH, _CH)],
                             ssem.at[par])

        for p in range(_NP - 1):
            @pl.when(p < nch)
            def _(p=p):
                fire(jnp.int32(p), p)

        def quad_step(cc, _):
            for par in range(_NP):
                j = _NP * cc + par
                @pl.when(j < nch)
                def _(j=j, par=par):
                    @pl.when(j + _NP - 1 < nch)
                    def _():
                        fire(j + _NP - 1, (par + _NP - 1) % _NP)
                    finish(j, par)
            return 0

        lax.fori_loop(0, (nch + _NP - 1) // _NP, quad_step, 0)
        for p in range(_NP):
            @pl.when(nch >= p + 1)
            def _(p=p):
                wait_out(p)

    @pl.when(row == 0)
    def _():
        run_span(text_emb, text_pos, jnp.int32(0), nfull)

    @pl.when(row == 1)
    def _():
        def zpair(cc, _):
            for par in range(2):
                j = 2 * cc + par
                @pl.when(j < nfull)
                def _():
                    @pl.when(j >= 2)
                    def _():
                        pltpu.make_async_copy(zrow_v, out_hbm.at[pl.ds(0, _CH)],
                                              ssem.at[par]).wait()
                    pltpu.async_copy(zrow_v,
                                     out_hbm.at[pl.ds(base + j * _CH, _CH)],
                                     ssem.at[par])
            return 0

        lax.fori_loop(0, (nfull + 1) // 2, zpair, 0)
        for par in range(2):
            @pl.when(nfull >= par + 1)
            def _():
                pltpu.make_async_copy(zrow_v, out_hbm.at[pl.ds(0, _CH)],
                                      ssem.at[par]).wait()

    @pl.when(rem > 0)
    def _():
        ci = nfull
        tk = pl.ds(row_base + ci * _CH, _CH)
        pk = pl.ds(ci * _CH, _CH)
        pltpu.async_copy(speech_emb.at[ids_v.at[tk]], ra_v.at[1], gsa.at[1])
        pltpu.async_copy(speech_pos.at[pos_v.at[pk]], rb_v.at[1], gsb.at[1])
        pltpu.make_async_copy(text_emb.at[pl.ds(0, _CH)], ra_v.at[1],
                              gsa.at[1]).wait()
        pltpu.make_async_copy(text_emb.at[pl.ds(0, _CH)], rb_v.at[1],
                              gsb.at[1]).wait()
        add_rows(1)

        @pl.when(row == 0)
        def _():
            pltpu.async_copy(text_emb.at[ids_v.at[tk]], ra_v.at[0], gsa.at[0])
            pltpu.async_copy(text_pos.at[pos_v.at[pk]], rb_v.at[0], gsb.at[0])
            pltpu.make_async_copy(text_emb.at[pl.ds(0, _CH)], ra_v.at[0],
                                  gsa.at[0]).wait()
            pltpu.make_async_copy(text_emb.at[pl.ds(0, _CH)], rb_v.at[0],
                                  gsb.at[0]).wait()
            add_rows(0)
            for r in range(_CH):
                @pl.when(r >= rem)
                def _(r=r):
                    def cp(i, _):
                        for u in range(_UNR):
                            d = pl.ds((i * _UNR + u) * _L, _L)
                            ob_v[0, r, d] = ob_v[1, r, d]
                        return 0
                    lax.fori_loop(0, _DIM // (_L * _UNR), cp, 0)

        @pl.when(row == 1)
        def _():
            for r in range(_CH):
                @pl.when(r < rem)
                def _(r=r):
                    def cp(i, _):
                        for u in range(_UNR):
                            d = pl.ds((i * _UNR + u) * _L, _L)
                            ob_v[1, r, d] = zrow_v[r, d]
                        return 0
                    lax.fori_loop(0, _DIM // (_L * _UNR), cp, 0)

        @pl.when(row == 0)
        def _():
            pltpu.async_copy(ob_v.at[0], out_hbm.at[pl.ds(base + ci * _CH, _CH)],
                             ssem.at[0])
            wait_out(0)
        @pl.when(row == 1)
        def _():
            pltpu.async_copy(ob_v.at[1], out_hbm.at[pl.ds(base + ci * _CH, _CH)],
                             ssem.at[1])
            wait_out(1)

    run_span(speech_emb, speech_pos, sb, jnp.int32(_NCHK))

    def exag_step(j, _):
        v = ids_v[pl.ds(row_base + j * _L, _L)]
        exm = jnp.where(v == _EXAG_TOK, 1, 0)
        any_v = exm
        for sh in (1, 2, 4, 8):
            any_v = jnp.maximum(any_v,
                                jnp.take_along_axis(any_v, iota ^ sh, axis=0))
        @pl.when(any_v[0] > 0)
        def _():
            for lane in range(_L):
                @pl.when(exm[lane] > 0)
                def _(lane=lane):
                    pltpu.sync_copy(
                        erow_v,
                        out_hbm.at[pl.ds(base + j * _L + lane, 1)])
        return 0

    lax.fori_loop(0, _TPW // _L, exag_step, 0)


@jax.jit
def _run(ids, pos, exag16, wcol, bias, text_emb, text_pos, speech_emb, speech_pos):
    n = ids.shape[0]
    mesh = plsc.VectorSubcoreMesh(core_axis_name="c", subcore_axis_name="s")
    f = functools.partial(
        pl.kernel, mesh=mesh,
        out_type=jax.ShapeDtypeStruct((n, _DIM), jnp.float32),
        scratch_types=[
            pltpu.VMEM((_NS * _TPW + _L,), jnp.int32),
            pltpu.VMEM((_TPW,), jnp.int32),
            pltpu.VMEM((_DIM,), jnp.float32),
            pltpu.VMEM((_DIM,), jnp.float32),
            pltpu.VMEM((_L,), jnp.float32),
            pltpu.VMEM((1, _DIM), jnp.float32),
            pltpu.VMEM((_CH, _DIM), jnp.float32),
            pltpu.VMEM((_NP, _CH, _DIM), jnp.float32),
            pltpu.VMEM((_NP, _CH, _DIM), jnp.float32),
            pltpu.VMEM((_NP, _CH, _DIM), jnp.float32),
            pltpu.SemaphoreType.DMA((_NP,)),
            pltpu.SemaphoreType.DMA((_NP,)),
            pltpu.SemaphoreType.DMA((_NP,)),
        ],
    )(_body)
    return f(ids, pos, exag16, wcol, bias, text_emb, text_pos, speech_emb, speech_pos)


def kernel(input_ids, position_ids, exaggeration, text_emb, text_pos_emb,
           speech_emb, speech_pos_emb, W_exag, b_exag):
    Bb, Ss = input_ids.shape
    ids = input_ids.reshape(-1).astype(jnp.int32)
    pos = position_ids.reshape(-1).astype(jnp.int32)
    exag16 = jnp.broadcast_to(exaggeration.astype(jnp.float32), (_L,))
    wcol = W_exag[:, 0].astype(jnp.float32)
    out = _run(ids, pos, exag16, wcol, b_exag.astype(jnp.float32),
               text_emb, text_pos_emb, speech_emb, speech_pos_emb)
    return out.reshape(Bb, Ss, _DIM)
